# baseline (speedup 1.0000x reference)
import jax
import jax.numpy as jnp
from jax import lax
from jax.experimental import pallas as pl
from jax.experimental.pallas import tpu as pltpu

B, T, D = 16, 4096, 300
DBLK = 50


def _tc_body(x_ref, xl_ref, o_ref):
    j = pl.program_id(1)
    xl = xl_ref[...][:, 0:1]
    tio = j * (T // 2) + lax.broadcasted_iota(jnp.int32, (B, T // 2), 1)
    mask = (tio < xl).astype(jnp.float32)
    o_ref[...] = x_ref[...] * mask[None, :, :]


def _masked_copy_tc(xt, xl2d):
    return pl.pallas_call(
        _tc_body,
        grid=(D // DBLK, 2),
        in_specs=[
            pl.BlockSpec((DBLK, B, T // 2), lambda i, j: (i, 0, j)),
            pl.BlockSpec((B, 128), lambda i, j: (0, 0)),
        ],
        out_specs=pl.BlockSpec((DBLK, B, T // 2), lambda i, j: (i, 0, j)),
        out_shape=jax.ShapeDtypeStruct((D, B, T), jnp.float32),
    )(xt, xl2d)


def kernel(x, x_len):
    xt = lax.transpose(x, (2, 0, 1))
    xl2d = jnp.broadcast_to(x_len.astype(jnp.int32)[:, None], (B, 128))
    out_t = _masked_copy_tc(xt, xl2d)
    return lax.transpose(out_t, (1, 2, 0))


# speedup vs baseline: 1.0270x; 1.0270x over previous
import jax
import jax.numpy as jnp
from jax import lax
from jax.experimental import pallas as pl
from jax.experimental.pallas import tpu as pltpu

B, T, D = 16, 4096, 300
DBLK = 50


def _tc_body(x_ref, xl_ref, o_ref):
    xl = xl_ref[...][:, 0:1]
    tio = lax.broadcasted_iota(jnp.int32, (B, T), 1)
    mask = (tio < xl).astype(jnp.float32)
    o_ref[...] = x_ref[...] * mask[None, :, :]


def _masked_copy_tc(xt, xl2d):
    return pl.pallas_call(
        _tc_body,
        grid=(D // DBLK,),
        in_specs=[
            pl.BlockSpec((DBLK, B, T), lambda i: (i, 0, 0)),
            pl.BlockSpec((B, 128), lambda i: (0, 0)),
        ],
        out_specs=pl.BlockSpec((DBLK, B, T), lambda i: (i, 0, 0)),
        out_shape=jax.ShapeDtypeStruct((D, B, T), jnp.float32),
    )(xt, xl2d)


def kernel(x, x_len):
    xt = lax.transpose(x, (2, 0, 1))
    xl2d = jnp.broadcast_to(x_len.astype(jnp.int32)[:, None], (B, 128))
    out_t = _masked_copy_tc(xt, xl2d)
    return lax.transpose(out_t, (1, 2, 0))
